# Initial kernel scaffold; baseline (speedup 1.0000x reference)
import jax
import jax.numpy as jnp
from jax.experimental import pallas as pl


def kernel(hidden_states, gate_w, w_gate, w_up, w_down, sh_gate, sh_up, sh_down):
    raise NotImplementedError("write your pallas kernel here")


# speedup vs baseline: 1.5680x; 1.5680x over previous
import functools

import jax
import jax.numpy as jnp
from jax.experimental import pallas as pl
from jax.experimental.pallas import tpu as pltpu

E = 8
TOP_K = 2
D_MODEL = 1024
D_FF = 704
T = 2048
TM = 256

_NT = (((1,), (1,)), ((), ()))


def _silu(x):
    return x * jax.nn.sigmoid(x)


def _moe_body(x_ref, gate_ref, wg_ref, wu_ref, wd_ref, shg_ref, shu_ref,
              shd_ref, out_ref):
    x = x_ref[...]

    logits = jax.lax.dot_general(
        x, gate_ref[...], _NT,
        preferred_element_type=jnp.float32)
    m = jnp.max(logits, axis=1, keepdims=True)
    ex = jnp.exp(logits - m)
    rw = ex / jnp.sum(ex, axis=1, keepdims=True)

    cols = [rw[:, e:e + 1] for e in range(E)]
    m1 = functools.reduce(jnp.maximum, cols)
    taken = [None] * E
    seen = jnp.zeros_like(m1, dtype=jnp.bool_)
    for e in range(E):
        hit = (cols[e] == m1) & (~seen)
        taken[e] = hit
        seen = seen | hit
    cols2 = [jnp.where(taken[e], -jnp.inf, cols[e]) for e in range(E)]
    m2 = functools.reduce(jnp.maximum, cols2)
    seen2 = jnp.zeros_like(m1, dtype=jnp.bool_)
    coef = [None] * E
    for e in range(E):
        hit2 = (cols2[e] == m2) & (~seen2)
        seen2 = seen2 | hit2
        coef[e] = jnp.where(taken[e] | hit2, cols[e], 0.0)

    xb = x.astype(jnp.bfloat16)
    acc = jnp.zeros((x.shape[0], D_MODEL), dtype=jnp.float32)
    for e in range(E):
        g = jax.lax.dot_general(xb, wg_ref[e], _NT,
                                preferred_element_type=jnp.float32)
        u = jax.lax.dot_general(xb, wu_ref[e], _NT,
                                preferred_element_type=jnp.float32)
        t = (_silu(g) * u).astype(jnp.bfloat16)
        y = jax.lax.dot_general(t, wd_ref[e], _NT,
                                preferred_element_type=jnp.float32)
        acc = acc + coef[e] * y

    gs = jax.lax.dot_general(xb, shg_ref[...], _NT,
                             preferred_element_type=jnp.float32)
    us = jax.lax.dot_general(xb, shu_ref[...], _NT,
                             preferred_element_type=jnp.float32)
    ts = (_silu(gs) * us).astype(jnp.bfloat16)
    sh = jax.lax.dot_general(ts, shd_ref[...], _NT,
                             preferred_element_type=jnp.float32)

    out_ref[...] = acc + sh


def kernel(hidden_states, gate_w, w_gate, w_up, w_down, sh_gate, sh_up,
           sh_down):
    bsz, seq_len, h = hidden_states.shape
    x = hidden_states.reshape(-1, h)
    d_sh = sh_gate.shape[0]

    wg = w_gate.astype(jnp.bfloat16)
    wu = w_up.astype(jnp.bfloat16)
    wd = w_down.astype(jnp.bfloat16)
    shg = sh_gate.astype(jnp.bfloat16)
    shu = sh_up.astype(jnp.bfloat16)
    shd = sh_down.astype(jnp.bfloat16)

    grid = (T // TM,)
    const = lambda shape: pl.BlockSpec(shape, lambda i: (0,) * len(shape))
    out = pl.pallas_call(
        _moe_body,
        grid=grid,
        in_specs=[
            pl.BlockSpec((TM, D_MODEL), lambda i: (i, 0)),
            const((E, D_MODEL)),
            const((E, D_FF, D_MODEL)),
            const((E, D_FF, D_MODEL)),
            const((E, D_MODEL, D_FF)),
            const((d_sh, D_MODEL)),
            const((d_sh, D_MODEL)),
            const((D_MODEL, d_sh)),
        ],
        out_specs=pl.BlockSpec((TM, D_MODEL), lambda i: (i, 0)),
        out_shape=jax.ShapeDtypeStruct((T, D_MODEL), jnp.float32),
    )(x, gate_w, wg, wu, wd, shg, shu, shd)
    return out.reshape(bsz, seq_len, h).astype(hidden_states.dtype)
